# Initial kernel scaffold; baseline (speedup 1.0000x reference)
import jax
import jax.numpy as jnp
from jax.experimental import pallas as pl


def kernel(codebook, audio_tokens, seq_lens):
    raise NotImplementedError("write your pallas kernel here")


# speedup vs baseline: 1.0304x; 1.0304x over previous
import functools

import jax
import jax.numpy as jnp
from jax import lax
from jax.experimental import pallas as pl
from jax.experimental.pallas import tpu as pltpu
from jax.experimental.pallas import tpu_sc as plsc

B = 64
L_OUT = 375
V = 16384
D = 1280
T_PAD = 384
N_TOK = B * T_PAD
NC = 2
NS = 16
NW = NC * NS
TOK_PER_W = N_TOK // NW
CHUNK = 64
N_CHUNKS = TOK_PER_W // CHUNK


def _sc_gather(codebook, tok_flat):
    mesh = plsc.VectorSubcoreMesh(core_axis_name="c", subcore_axis_name="s")

    @functools.partial(
        pl.kernel,
        out_type=jax.ShapeDtypeStruct((N_TOK, D), jnp.float32),
        mesh=mesh,
        scratch_types=[
            pltpu.VMEM((CHUNK,), jnp.int32),
            pltpu.VMEM((CHUNK, D), jnp.float32),
            pltpu.SemaphoreType.DMA,
        ],
    )
    def gather_kernel(table_hbm, idx_hbm, out_hbm, idx_v, rows_v, sem):
        wid = lax.axis_index("s") * NC + lax.axis_index("c")
        base = wid * TOK_PER_W

        @pl.loop(0, N_CHUNKS)
        def _(c):
            start = base + c * CHUNK
            pltpu.sync_copy(idx_hbm.at[pl.ds(start, CHUNK)], idx_v)
            pltpu.async_copy(table_hbm.at[idx_v], rows_v, sem).wait()
            pltpu.sync_copy(rows_v, out_hbm.at[pl.ds(start, CHUNK)])

    return gather_kernel(codebook, tok_flat)


def _tc_transpose_mask(gathered, seq_lens):
    def body(seq_ref, x_ref, o_ref):
        b = pl.program_id(0)
        x = x_ref[0]
        xt = x.T
        t_iota = lax.broadcasted_iota(jnp.int32, (D, T_PAD), 1)
        y = jnp.where(t_iota < seq_ref[b], xt, 0.0)
        o_ref[0] = y[:, :L_OUT]

    return pl.pallas_call(
        body,
        grid=(B,),
        in_specs=[
            pl.BlockSpec(memory_space=pltpu.SMEM),
            pl.BlockSpec((1, T_PAD, D), lambda b: (b, 0, 0)),
        ],
        out_specs=pl.BlockSpec((1, D, L_OUT), lambda b: (b, 0, 0)),
        out_shape=jax.ShapeDtypeStruct((B, D, L_OUT), jnp.float32),
        compiler_params=pltpu.CompilerParams(
            dimension_semantics=("parallel",),
        ),
    )(seq_lens, gathered.reshape(B, T_PAD, D))


def kernel(codebook, audio_tokens, seq_lens):
    tok_flat = jnp.pad(audio_tokens, ((0, 0), (0, T_PAD - L_OUT))).reshape(-1)
    gathered = _sc_gather(codebook, tok_flat)
    out = _tc_transpose_mask(gathered, seq_lens)
    return (out, seq_lens)
